# Initial kernel scaffold; baseline (speedup 1.0000x reference)
import jax
import jax.numpy as jnp
from jax.experimental import pallas as pl


def kernel(z, r, table):
    raise NotImplementedError("write your pallas kernel here")


# speedup vs baseline: 178.3203x; 178.3203x over previous
import functools

import jax
import jax.numpy as jnp
from jax import lax
from jax.experimental import pallas as pl
from jax.experimental.pallas import tpu as pltpu
from jax.experimental.pallas import tpu_sc as plsc

N = 2_000_000
NUM_WORKERS = 32
CHUNK = 62_496
TAIL = N - NUM_WORKERS * CHUNK
TABLE_PAD = 128
LANES = 16

_mesh = plsc.VectorSubcoreMesh(core_axis_name="c", subcore_axis_name="s")


@functools.partial(
    pl.kernel,
    out_type=jax.ShapeDtypeStruct((N,), jnp.int32),
    mesh=_mesh,
    compiler_params=pltpu.CompilerParams(needs_layout_passes=False),
    scratch_types=[
        pltpu.VMEM((TABLE_PAD,), jnp.int32),
        pltpu.VMEM((CHUNK,), jnp.int32),
        pltpu.VMEM((TAIL,), jnp.int32),
    ],
)
def _translate(z_hbm, table_hbm, out_hbm, table_v, buf_v, tail_v):
    wid = lax.axis_index("s") * 2 + lax.axis_index("c")
    base = wid * CHUNK

    pltpu.sync_copy(table_hbm, table_v)
    pltpu.sync_copy(z_hbm.at[pl.ds(base, CHUNK)], buf_v)

    def body(i, carry):
        idx = buf_v[pl.ds(i * LANES, LANES)]
        buf_v[pl.ds(i * LANES, LANES)] = plsc.load_gather(table_v, [idx])
        return carry

    lax.fori_loop(0, CHUNK // LANES, body, 0)
    pltpu.sync_copy(buf_v, out_hbm.at[pl.ds(base, CHUNK)])

    @pl.when(wid == 0)
    def _():
        tail_base = NUM_WORKERS * CHUNK
        pltpu.sync_copy(z_hbm.at[pl.ds(tail_base, TAIL)], tail_v)

        def tbody(i, carry):
            idx = tail_v[pl.ds(i * LANES, LANES)]
            tail_v[pl.ds(i * LANES, LANES)] = plsc.load_gather(table_v, [idx])
            return carry

        lax.fori_loop(0, TAIL // LANES, tbody, 0)
        pltpu.sync_copy(tail_v, out_hbm.at[pl.ds(tail_base, TAIL)])


def kernel(z, r, table):
    table_padded = jnp.pad(table, (0, TABLE_PAD - table.shape[0]))
    new_z = _translate(z, table_padded)
    return (new_z, r)


# speedup vs baseline: 235.7556x; 1.3221x over previous
import functools

import jax
import jax.numpy as jnp
from jax import lax
from jax.experimental import pallas as pl
from jax.experimental.pallas import tpu as pltpu
from jax.experimental.pallas import tpu_sc as plsc

N = 2_000_000
NUM_WORKERS = 32
LANES = 16
UNROLL = 8
STEP = LANES * UNROLL
CHUNK = 62_464
TAIL = N - NUM_WORKERS * CHUNK
TABLE_PAD = 128

_mesh = plsc.VectorSubcoreMesh(core_axis_name="c", subcore_axis_name="s")


@functools.partial(
    pl.kernel,
    out_type=jax.ShapeDtypeStruct((N,), jnp.int32),
    mesh=_mesh,
    compiler_params=pltpu.CompilerParams(needs_layout_passes=False),
    scratch_types=[
        pltpu.VMEM((TABLE_PAD,), jnp.int32),
        pltpu.VMEM((CHUNK,), jnp.int32),
        pltpu.VMEM((CHUNK,), jnp.int32),
        pltpu.VMEM((TAIL,), jnp.int32),
    ],
)
def _translate(z_hbm, table_hbm, out_hbm, table_v, in_v, out_v, tail_v):
    wid = lax.axis_index("s") * 2 + lax.axis_index("c")
    base = wid * CHUNK

    pltpu.sync_copy(table_hbm, table_v)
    pltpu.sync_copy(z_hbm.at[pl.ds(base, CHUNK)], in_v)

    def body(i, carry):
        off = i * STEP
        for j in range(UNROLL):
            idx = in_v[pl.ds(off + j * LANES, LANES)]
            out_v[pl.ds(off + j * LANES, LANES)] = plsc.load_gather(
                table_v, [idx]
            )
        return carry

    lax.fori_loop(0, CHUNK // STEP, body, 0)
    pltpu.sync_copy(out_v, out_hbm.at[pl.ds(base, CHUNK)])

    @pl.when(wid == 0)
    def _():
        tail_base = NUM_WORKERS * CHUNK
        pltpu.sync_copy(z_hbm.at[pl.ds(tail_base, TAIL)], tail_v)

        def tbody(i, carry):
            off = i * STEP
            for j in range(UNROLL):
                idx = tail_v[pl.ds(off + j * LANES, LANES)]
                tail_v[pl.ds(off + j * LANES, LANES)] = plsc.load_gather(
                    table_v, [idx]
                )
            return carry

        lax.fori_loop(0, TAIL // STEP, tbody, 0)
        pltpu.sync_copy(tail_v, out_hbm.at[pl.ds(tail_base, TAIL)])


def kernel(z, r, table):
    table_padded = jnp.pad(table, (0, TABLE_PAD - table.shape[0]))
    new_z = _translate(z, table_padded)
    return (new_z, r)
